# Initial kernel scaffold; baseline (speedup 1.0000x reference)
import jax
import jax.numpy as jnp
from jax.experimental import pallas as pl


def kernel(hyperedge, hyper_node, ve_affiliation, W_v2e, b_v2e, W_k, b_k, W_v, b_v, att_r, ln0_g, ln0_b, W_rff, b_rff, ln1_g, ln1_b, W_upd, b_upd):
    raise NotImplementedError("write your pallas kernel here")


# speedup vs baseline: 56.0017x; 56.0017x over previous
import functools

import jax
import jax.numpy as jnp
from jax import lax
from jax.experimental import pallas as pl
from jax.experimental.pallas import tpu as pltpu
from jax.experimental.pallas import tpu_sc as plsc

E = 320000
NUM_HE = 10000
D = 128
HEADS = 8
D_H = 16

BE = 2000
RB = 1000
CHUNK = 128
IDXR = CHUNK // 128
N_CHUNKS = E // CHUNK
CH_PER_W = N_CHUNKS // 32
CH_TAIL = N_CHUNKS - 32 * CH_PER_W
NHE_PAD = 10240
ROWS_PER_TILE = NHE_PAD // 16
CLAMP = 70.0


def _edge_body(x_ref, w1_ref, b1_ref, wv_ref, bv_ref, wa_ref, ba_ref,
               w_out_ref, ex_out_ref):
    x = x_ref[...]
    h = jnp.maximum(
        jnp.dot(x, w1_ref[...], preferred_element_type=jnp.float32) + b1_ref[...], 0.0)
    v = jnp.dot(h, wv_ref[...], preferred_element_type=jnp.float32) + bv_ref[...]
    a = jnp.dot(h, wa_ref[...], preferred_element_type=jnp.float32) + ba_ref[...]
    e = jnp.exp(jnp.minimum(a, CLAMP))
    w_out_ref[...] = e * v
    ex_out_ref[...] = e


def _post_body(aggp_ref, denp_ref, he_ref, att_ref, g0_ref, be0_ref,
               wr_ref, br_ref, g1_ref, be1_ref, wut_ref, wub_ref, bu_ref,
               out_ref):
    hi = jax.lax.Precision.HIGHEST
    aggp = aggp_ref[...]
    denp = denp_ref[...]
    agg = aggp[0] + aggp[1]
    den = denp[0] + denp[1]
    out = agg / (den + 1e-16) + att_ref[...]
    mu = jnp.mean(out, axis=-1, keepdims=True)
    var = jnp.mean((out - mu) ** 2, axis=-1, keepdims=True)
    out = (out - mu) * lax.rsqrt(var + 1e-5) * g0_ref[...] + be0_ref[...]
    rff = jnp.maximum(
        jnp.dot(out, wr_ref[...], preferred_element_type=jnp.float32,
                precision=hi) + br_ref[...], 0.0)
    y = out + rff
    mu2 = jnp.mean(y, axis=-1, keepdims=True)
    var2 = jnp.mean((y - mu2) ** 2, axis=-1, keepdims=True)
    y = (y - mu2) * lax.rsqrt(var2 + 1e-5) * g1_ref[...] + be1_ref[...]
    tem = jnp.maximum(y, 0.0)
    fin = (jnp.dot(tem, wut_ref[...], preferred_element_type=jnp.float32,
                   precision=hi)
           + jnp.dot(he_ref[...], wub_ref[...], preferred_element_type=jnp.float32,
                     precision=hi)
           + bu_ref[...])
    out_ref[...] = jnp.maximum(fin, 0.0)


def _sc_scatter_agg(w_hbm, idx_hbm, agg_out, w_v, idx_v, agg_sh):
    cid = lax.axis_index("c")
    sid = lax.axis_index("s")
    wid = cid * 16 + sid

    zeros16 = jnp.zeros((16,), jnp.float32)

    def zrow(r, carry):
        for c in range(8):
            w_v[r, pl.ds(c * 16, 16)] = zeros16
        return carry
    lax.fori_loop(0, 128, zrow, 0)

    base = sid * ROWS_PER_TILE
    for k in range(5):
        pltpu.sync_copy(w_v, agg_sh.at[pl.ds(base + k * 128, 128)])
    plsc.subcore_barrier()

    def do_chunk(ch):
        pltpu.sync_copy(w_hbm.at[pl.ds(ch * CHUNK, CHUNK)], w_v)
        pltpu.sync_copy(idx_hbm.at[ch], idx_v)
        pltpu.sync_copy(w_v, agg_sh.at[idx_v.at[0]], add=True)

    def body(k, carry):
        do_chunk(wid * CH_PER_W + k)
        return carry
    lax.fori_loop(0, CH_PER_W, body, 0)

    @pl.when(wid < CH_TAIL)
    def _tail():
        do_chunk(32 * CH_PER_W + wid)

    plsc.subcore_barrier()
    ob = sid * ROWS_PER_TILE
    pltpu.sync_copy(agg_sh.at[pl.ds(ob, ROWS_PER_TILE)],
                    agg_out.at[pl.ds(cid * NHE_PAD + ob, ROWS_PER_TILE)])


def _run_sc_segment_sum(w, ex, idx3d):
    mesh = plsc.VectorSubcoreMesh(core_axis_name="c", subcore_axis_name="s")
    agg_flat = functools.partial(
        pl.kernel,
        mesh=mesh,
        out_type=jax.ShapeDtypeStruct((2 * NHE_PAD, 128), jnp.float32),
        scratch_types=[
            pltpu.VMEM((CHUNK, 128), jnp.float32),
            pltpu.VMEM((IDXR, 128), jnp.int32),
            pltpu.VMEM_SHARED((NHE_PAD, 128), jnp.float32),
        ],
    )(_sc_scatter_agg)(w, idx3d)
    idx3d_dep, agg_flat = lax.optimization_barrier((idx3d, agg_flat))
    den_flat = functools.partial(
        pl.kernel,
        mesh=mesh,
        out_type=jax.ShapeDtypeStruct((2 * NHE_PAD, 128), jnp.float32),
        scratch_types=[
            pltpu.VMEM((CHUNK, 128), jnp.float32),
            pltpu.VMEM((IDXR, 128), jnp.int32),
            pltpu.VMEM_SHARED((NHE_PAD, 128), jnp.float32),
        ],
    )(_sc_scatter_agg)(ex, idx3d_dep)
    return (agg_flat.reshape(2, NHE_PAD, 128), den_flat.reshape(2, NHE_PAD, 128))


def kernel(hyperedge, hyper_node, ve_affiliation, W_v2e, b_v2e, W_k, b_k,
           W_v, b_v, att_r, ln0_g, ln0_b, W_rff, b_rff, ln1_g, ln1_b,
           W_upd, b_upd):
    f32 = jnp.float32
    dst3d = ve_affiliation[0].reshape(N_CHUNKS, IDXR, 128)

    attm = att_r.reshape(HEADS, D_H)
    P = (jnp.eye(HEADS, dtype=f32)[:, None, :] * attm[:, :, None]).reshape(D, HEADS)
    scale = 1.0 / jnp.sqrt(jnp.float32(D_H))
    Wa8 = (W_k @ P) * scale
    ba8 = (b_k @ P) * scale
    Wa = jnp.repeat(Wa8, D_H, axis=1)
    ba = jnp.repeat(ba8, D_H).reshape(1, D)

    b1r = b_v2e.reshape(1, D)
    bvr = b_v.reshape(1, D)
    att_row = att_r.reshape(1, D)

    full = lambda shape: pl.BlockSpec(shape, lambda i: tuple(0 for _ in shape))
    W, EX = pl.pallas_call(
        _edge_body,
        grid=(E // BE,),
        in_specs=[
            pl.BlockSpec((BE, D), lambda i: (i, 0)),
            full((D, D)), full((1, D)),
            full((D, D)), full((1, D)),
            full((D, D)), full((1, D)),
        ],
        out_specs=[
            pl.BlockSpec((BE, D), lambda i: (i, 0)),
            pl.BlockSpec((BE, D), lambda i: (i, 0)),
        ],
        out_shape=[
            jax.ShapeDtypeStruct((E, D), f32),
            jax.ShapeDtypeStruct((E, D), f32),
        ],
    )(hyper_node, W_v2e, b1r, W_v, bvr, Wa, ba)

    aggp, denp = _run_sc_segment_sum(W, EX, dst3d)

    out = pl.pallas_call(
        _post_body,
        grid=(NUM_HE // RB,),
        in_specs=[
            pl.BlockSpec((2, RB, D), lambda i: (0, i, 0)),
            pl.BlockSpec((2, RB, D), lambda i: (0, i, 0)),
            pl.BlockSpec((RB, D), lambda i: (i, 0)),
            full((1, D)),
            full((1, D)), full((1, D)),
            full((D, D)), full((1, D)),
            full((1, D)), full((1, D)),
            full((D, D)), full((D, D)), full((1, D)),
        ],
        out_specs=pl.BlockSpec((RB, D), lambda i: (i, 0)),
        out_shape=jax.ShapeDtypeStruct((NUM_HE, D), f32),
    )(aggp, denp, hyperedge, att_row,
      ln0_g.reshape(1, D), ln0_b.reshape(1, D),
      W_rff, b_rff.reshape(1, D),
      ln1_g.reshape(1, D), ln1_b.reshape(1, D),
      W_upd[:D], W_upd[D:], b_upd.reshape(1, D))
    return out


# speedup vs baseline: 60.5664x; 1.0815x over previous
import functools

import jax
import jax.numpy as jnp
from jax import lax
from jax.experimental import pallas as pl
from jax.experimental.pallas import tpu as pltpu
from jax.experimental.pallas import tpu_sc as plsc

E = 320000
NUM_HE = 10000
D = 128
HEADS = 8
D_H = 16

BE = 2000
RB = 1000
CHUNK = 128
IDXR = CHUNK // 128
N_CHUNKS = E // CHUNK
GSZ = 8
N_GROUPS = N_CHUNKS // GSZ
CH_TAILC = N_CHUNKS - N_GROUPS * GSZ
G_PER_W = N_GROUPS // 32
G_HI = N_GROUPS - 32 * G_PER_W
NHE_PAD = 10240
ROWS_PER_TILE = NHE_PAD // 16
CLAMP = 70.0


def _edge_body(x_ref, w1_ref, b1_ref, wv_ref, bv_ref, wa_ref, ba_ref,
               w_out_ref, ex_out_ref):
    x = x_ref[...]
    h = jnp.maximum(
        jnp.dot(x, w1_ref[...], preferred_element_type=jnp.float32) + b1_ref[...], 0.0)
    v = jnp.dot(h, wv_ref[...], preferred_element_type=jnp.float32) + bv_ref[...]
    a = jnp.dot(h, wa_ref[...], preferred_element_type=jnp.float32) + ba_ref[...]
    e = jnp.exp(jnp.minimum(a, CLAMP))
    w_out_ref[...] = e * v
    ex_out_ref[...] = e


def _post_body(aggp_ref, denp_ref, he_ref, att_ref, g0_ref, be0_ref,
               wr_ref, br_ref, g1_ref, be1_ref, wut_ref, wub_ref, bu_ref,
               out_ref):
    hi = jax.lax.Precision.HIGHEST
    aggp = aggp_ref[...]
    denp = denp_ref[...]
    agg = aggp[0] + aggp[1]
    den = denp[0] + denp[1]
    out = agg / (den + 1e-16) + att_ref[...]
    mu = jnp.mean(out, axis=-1, keepdims=True)
    var = jnp.mean((out - mu) ** 2, axis=-1, keepdims=True)
    out = (out - mu) * lax.rsqrt(var + 1e-5) * g0_ref[...] + be0_ref[...]
    rff = jnp.maximum(
        jnp.dot(out, wr_ref[...], preferred_element_type=jnp.float32,
                precision=hi) + br_ref[...], 0.0)
    y = out + rff
    mu2 = jnp.mean(y, axis=-1, keepdims=True)
    var2 = jnp.mean((y - mu2) ** 2, axis=-1, keepdims=True)
    y = (y - mu2) * lax.rsqrt(var2 + 1e-5) * g1_ref[...] + be1_ref[...]
    tem = jnp.maximum(y, 0.0)
    fin = (jnp.dot(tem, wut_ref[...], preferred_element_type=jnp.float32,
                   precision=hi)
           + jnp.dot(he_ref[...], wub_ref[...], preferred_element_type=jnp.float32,
                     precision=hi)
           + bu_ref[...])
    out_ref[...] = jnp.maximum(fin, 0.0)


def _sc_scatter_agg(w_hbm, idxg_hbm, idxt_hbm, agg_out, w_v, idx_v, agg_sh):
    cid = lax.axis_index("c")
    sid = lax.axis_index("s")
    wid = cid * 16 + sid

    zeros16 = jnp.zeros((16,), jnp.float32)

    def zrow(r, carry):
        for c in range(8):
            w_v[r, pl.ds(c * 16, 16)] = zeros16
        return carry
    lax.fori_loop(0, 128, zrow, 0)

    base = sid * ROWS_PER_TILE
    for k in range(5):
        pltpu.sync_copy(w_v, agg_sh.at[pl.ds(base + k * 128, 128)])
    plsc.subcore_barrier()

    def do_group(g):
        pltpu.sync_copy(idxg_hbm.at[g], idx_v)
        for j in range(GSZ):
            pltpu.sync_copy(
                w_hbm.at[pl.ds((g * GSZ + j) * CHUNK, CHUNK)], w_v)
            pltpu.sync_copy(w_v, agg_sh.at[idx_v.at[j]], add=True)

    gbase = wid * G_PER_W + jnp.minimum(wid, G_HI)

    def body(k, carry):
        do_group(gbase + k)
        return carry
    lax.fori_loop(0, G_PER_W, body, 0)

    @pl.when(wid < G_HI)
    def _extra():
        do_group(gbase + G_PER_W)

    @pl.when(wid < CH_TAILC)
    def _tail():
        ch = N_GROUPS * GSZ + wid
        pltpu.sync_copy(idxt_hbm.at[ch], idx_v.at[pl.ds(0, 1)])
        pltpu.sync_copy(w_hbm.at[pl.ds(ch * CHUNK, CHUNK)], w_v)
        pltpu.sync_copy(w_v, agg_sh.at[idx_v.at[0]], add=True)

    plsc.subcore_barrier()
    ob = sid * ROWS_PER_TILE
    pltpu.sync_copy(agg_sh.at[pl.ds(ob, ROWS_PER_TILE)],
                    agg_out.at[pl.ds(cid * NHE_PAD + ob, ROWS_PER_TILE)])


def _run_sc_segment_sum(w, ex, idx3d, idxg):
    mesh = plsc.VectorSubcoreMesh(core_axis_name="c", subcore_axis_name="s")
    agg_flat = functools.partial(
        pl.kernel,
        mesh=mesh,
        out_type=jax.ShapeDtypeStruct((2 * NHE_PAD, 128), jnp.float32),
        scratch_types=[
            pltpu.VMEM((CHUNK, 128), jnp.float32),
            pltpu.VMEM((GSZ, 128), jnp.int32),
            pltpu.VMEM_SHARED((NHE_PAD, 128), jnp.float32),
        ],
    )(_sc_scatter_agg)(w, idxg, idx3d)
    idx3d_dep, idxg_dep, agg_flat = lax.optimization_barrier(
        (idx3d, idxg, agg_flat))
    den_flat = functools.partial(
        pl.kernel,
        mesh=mesh,
        out_type=jax.ShapeDtypeStruct((2 * NHE_PAD, 128), jnp.float32),
        scratch_types=[
            pltpu.VMEM((CHUNK, 128), jnp.float32),
            pltpu.VMEM((GSZ, 128), jnp.int32),
            pltpu.VMEM_SHARED((NHE_PAD, 128), jnp.float32),
        ],
    )(_sc_scatter_agg)(ex, idxg_dep, idx3d_dep)
    return (agg_flat.reshape(2, NHE_PAD, 128), den_flat.reshape(2, NHE_PAD, 128))


def kernel(hyperedge, hyper_node, ve_affiliation, W_v2e, b_v2e, W_k, b_k,
           W_v, b_v, att_r, ln0_g, ln0_b, W_rff, b_rff, ln1_g, ln1_b,
           W_upd, b_upd):
    f32 = jnp.float32
    dst = ve_affiliation[0]
    dst3d = dst.reshape(N_CHUNKS, IDXR, 128)
    dstg = dst[: N_GROUPS * GSZ * CHUNK].reshape(N_GROUPS, GSZ, 128)

    attm = att_r.reshape(HEADS, D_H)
    P = (jnp.eye(HEADS, dtype=f32)[:, None, :] * attm[:, :, None]).reshape(D, HEADS)
    scale = 1.0 / jnp.sqrt(jnp.float32(D_H))
    Wa8 = (W_k @ P) * scale
    ba8 = (b_k @ P) * scale
    Wa = jnp.repeat(Wa8, D_H, axis=1)
    ba = jnp.repeat(ba8, D_H).reshape(1, D)

    b1r = b_v2e.reshape(1, D)
    bvr = b_v.reshape(1, D)
    att_row = att_r.reshape(1, D)

    full = lambda shape: pl.BlockSpec(shape, lambda i: tuple(0 for _ in shape))
    W, EX = pl.pallas_call(
        _edge_body,
        grid=(E // BE,),
        in_specs=[
            pl.BlockSpec((BE, D), lambda i: (i, 0)),
            full((D, D)), full((1, D)),
            full((D, D)), full((1, D)),
            full((D, D)), full((1, D)),
        ],
        out_specs=[
            pl.BlockSpec((BE, D), lambda i: (i, 0)),
            pl.BlockSpec((BE, D), lambda i: (i, 0)),
        ],
        out_shape=[
            jax.ShapeDtypeStruct((E, D), f32),
            jax.ShapeDtypeStruct((E, D), f32),
        ],
    )(hyper_node, W_v2e, b1r, W_v, bvr, Wa, ba)

    aggp, denp = _run_sc_segment_sum(W, EX, dst3d, dstg)

    out = pl.pallas_call(
        _post_body,
        grid=(NUM_HE // RB,),
        in_specs=[
            pl.BlockSpec((2, RB, D), lambda i: (0, i, 0)),
            pl.BlockSpec((2, RB, D), lambda i: (0, i, 0)),
            pl.BlockSpec((RB, D), lambda i: (i, 0)),
            full((1, D)),
            full((1, D)), full((1, D)),
            full((D, D)), full((1, D)),
            full((1, D)), full((1, D)),
            full((D, D)), full((D, D)), full((1, D)),
        ],
        out_specs=pl.BlockSpec((RB, D), lambda i: (i, 0)),
        out_shape=jax.ShapeDtypeStruct((NUM_HE, D), f32),
    )(aggp, denp, hyperedge, att_row,
      ln0_g.reshape(1, D), ln0_b.reshape(1, D),
      W_rff, b_rff.reshape(1, D),
      ln1_g.reshape(1, D), ln1_b.reshape(1, D),
      W_upd[:D], W_upd[D:], b_upd.reshape(1, D))
    return out


# speedup vs baseline: 65.6762x; 1.0844x over previous
import functools

import jax
import jax.numpy as jnp
from jax import lax
from jax.experimental import pallas as pl
from jax.experimental.pallas import tpu as pltpu
from jax.experimental.pallas import tpu_sc as plsc

E = 320000
NUM_HE = 10000
D = 128
HEADS = 8
D_H = 16

BE = 2000
RB = 1000
CHUNK = 128
IDXR = CHUNK // 128
N_CHUNKS = E // CHUNK
SCH = 64
N_SCH = E // SCH
GSZ = 8
N_GROUPS = N_SCH // GSZ
G_PER_W = N_GROUPS // 32
G_HI = N_GROUPS - 32 * G_PER_W
NHE_PAD = 10240
ROWS_PER_TILE = NHE_PAD // 16
CLAMP = 70.0


def _edge_body(x_ref, w1_ref, b1_ref, wv_ref, bv_ref, wa_ref, ba_ref,
               w_out_ref, ex_out_ref):
    x = x_ref[...]
    h = jnp.maximum(
        jnp.dot(x, w1_ref[...], preferred_element_type=jnp.float32) + b1_ref[...], 0.0)
    v = jnp.dot(h, wv_ref[...], preferred_element_type=jnp.float32) + bv_ref[...]
    a = jnp.dot(h, wa_ref[...], preferred_element_type=jnp.float32) + ba_ref[...]
    e = jnp.exp(jnp.minimum(a, CLAMP))
    w_out_ref[...] = e * v
    ex_out_ref[...] = e


def _post_body(aggp_ref, denp_ref, he_ref, att_ref, g0_ref, be0_ref,
               wr_ref, br_ref, g1_ref, be1_ref, wut_ref, wub_ref, bu_ref,
               out_ref):
    hi = jax.lax.Precision.HIGHEST
    aggp = aggp_ref[...]
    denp = denp_ref[...]
    agg = aggp[0] + aggp[1]
    den = denp[0] + denp[1]
    out = agg / (den + 1e-16) + att_ref[...]
    mu = jnp.mean(out, axis=-1, keepdims=True)
    var = jnp.mean((out - mu) ** 2, axis=-1, keepdims=True)
    out = (out - mu) * lax.rsqrt(var + 1e-5) * g0_ref[...] + be0_ref[...]
    rff = jnp.maximum(
        jnp.dot(out, wr_ref[...], preferred_element_type=jnp.float32,
                precision=hi) + br_ref[...], 0.0)
    y = out + rff
    mu2 = jnp.mean(y, axis=-1, keepdims=True)
    var2 = jnp.mean((y - mu2) ** 2, axis=-1, keepdims=True)
    y = (y - mu2) * lax.rsqrt(var2 + 1e-5) * g1_ref[...] + be1_ref[...]
    tem = jnp.maximum(y, 0.0)
    fin = (jnp.dot(tem, wut_ref[...], preferred_element_type=jnp.float32,
                   precision=hi)
           + jnp.dot(he_ref[...], wub_ref[...], preferred_element_type=jnp.float32,
                     precision=hi)
           + bu_ref[...])
    out_ref[...] = jnp.maximum(fin, 0.0)


def _sc_scatter_agg(w_hbm, idxg_hbm, agg_out,
                    w_v0, w_v1, idx_v, sem0, sem1, agg_sh):
    cid = lax.axis_index("c")
    sid = lax.axis_index("s")
    wid = cid * 16 + sid

    zeros16 = jnp.zeros((16,), jnp.float32)

    def zrow(r, carry):
        for c in range(8):
            w_v0[r, pl.ds(c * 16, 16)] = zeros16
        return carry
    lax.fori_loop(0, SCH, zrow, 0)

    base = sid * ROWS_PER_TILE
    for k in range(ROWS_PER_TILE // SCH):
        pltpu.sync_copy(w_v0, agg_sh.at[pl.ds(base + k * SCH, SCH)])
    plsc.subcore_barrier()

    bufs = (w_v0, w_v1)
    sems = (sem0, sem1)

    def do_group(g):
        pltpu.sync_copy(idxg_hbm.at[g], idx_v)
        c0 = g * GSZ
        pending = pltpu.async_copy(
            w_hbm.at[pl.ds(c0 * SCH, SCH)], w_v0, sem0)
        for j in range(GSZ):
            if j + 1 < GSZ:
                nxt = pltpu.async_copy(
                    w_hbm.at[pl.ds((c0 + j + 1) * SCH, SCH)],
                    bufs[(j + 1) % 2], sems[(j + 1) % 2])
            pending.wait()
            pltpu.sync_copy(bufs[j % 2], agg_sh.at[idx_v.at[j]], add=True)
            if j + 1 < GSZ:
                pending = nxt

    gbase = wid * G_PER_W + jnp.minimum(wid, G_HI)

    def body(k, carry):
        do_group(gbase + k)
        return carry
    lax.fori_loop(0, G_PER_W, body, 0)

    @pl.when(wid < G_HI)
    def _extra():
        do_group(gbase + G_PER_W)

    plsc.subcore_barrier()
    ob = sid * ROWS_PER_TILE
    pltpu.sync_copy(agg_sh.at[pl.ds(ob, ROWS_PER_TILE)],
                    agg_out.at[pl.ds(cid * NHE_PAD + ob, ROWS_PER_TILE)])


def _run_sc_segment_sum(w, ex, idxg):
    mesh = plsc.VectorSubcoreMesh(core_axis_name="c", subcore_axis_name="s")
    agg_flat = functools.partial(
        pl.kernel,
        mesh=mesh,
        out_type=jax.ShapeDtypeStruct((2 * NHE_PAD, 128), jnp.float32),
        scratch_types=[
            pltpu.VMEM((SCH, 128), jnp.float32),
            pltpu.VMEM((SCH, 128), jnp.float32),
            pltpu.VMEM((GSZ, SCH), jnp.int32),
            pltpu.SemaphoreType.DMA,
            pltpu.SemaphoreType.DMA,
            pltpu.VMEM_SHARED((NHE_PAD, 128), jnp.float32),
        ],
    )(_sc_scatter_agg)(w, idxg)
    idxg_dep, agg_flat = lax.optimization_barrier((idxg, agg_flat))
    den_flat = functools.partial(
        pl.kernel,
        mesh=mesh,
        out_type=jax.ShapeDtypeStruct((2 * NHE_PAD, 128), jnp.float32),
        scratch_types=[
            pltpu.VMEM((SCH, 128), jnp.float32),
            pltpu.VMEM((SCH, 128), jnp.float32),
            pltpu.VMEM((GSZ, SCH), jnp.int32),
            pltpu.SemaphoreType.DMA,
            pltpu.SemaphoreType.DMA,
            pltpu.VMEM_SHARED((NHE_PAD, 128), jnp.float32),
        ],
    )(_sc_scatter_agg)(ex, idxg_dep)
    return (agg_flat.reshape(2, NHE_PAD, 128), den_flat.reshape(2, NHE_PAD, 128))


def kernel(hyperedge, hyper_node, ve_affiliation, W_v2e, b_v2e, W_k, b_k,
           W_v, b_v, att_r, ln0_g, ln0_b, W_rff, b_rff, ln1_g, ln1_b,
           W_upd, b_upd):
    f32 = jnp.float32
    dstg = ve_affiliation[0].reshape(N_GROUPS, GSZ, SCH)

    attm = att_r.reshape(HEADS, D_H)
    P = (jnp.eye(HEADS, dtype=f32)[:, None, :] * attm[:, :, None]).reshape(D, HEADS)
    scale = 1.0 / jnp.sqrt(jnp.float32(D_H))
    Wa8 = (W_k @ P) * scale
    ba8 = (b_k @ P) * scale
    Wa = jnp.repeat(Wa8, D_H, axis=1)
    ba = jnp.repeat(ba8, D_H).reshape(1, D)

    b1r = b_v2e.reshape(1, D)
    bvr = b_v.reshape(1, D)
    att_row = att_r.reshape(1, D)

    full = lambda shape: pl.BlockSpec(shape, lambda i: tuple(0 for _ in shape))
    W, EX = pl.pallas_call(
        _edge_body,
        grid=(E // BE,),
        in_specs=[
            pl.BlockSpec((BE, D), lambda i: (i, 0)),
            full((D, D)), full((1, D)),
            full((D, D)), full((1, D)),
            full((D, D)), full((1, D)),
        ],
        out_specs=[
            pl.BlockSpec((BE, D), lambda i: (i, 0)),
            pl.BlockSpec((BE, D), lambda i: (i, 0)),
        ],
        out_shape=[
            jax.ShapeDtypeStruct((E, D), f32),
            jax.ShapeDtypeStruct((E, D), f32),
        ],
    )(hyper_node, W_v2e, b1r, W_v, bvr, Wa, ba)

    aggp, denp = _run_sc_segment_sum(W, EX, dstg)

    out = pl.pallas_call(
        _post_body,
        grid=(NUM_HE // RB,),
        in_specs=[
            pl.BlockSpec((2, RB, D), lambda i: (0, i, 0)),
            pl.BlockSpec((2, RB, D), lambda i: (0, i, 0)),
            pl.BlockSpec((RB, D), lambda i: (i, 0)),
            full((1, D)),
            full((1, D)), full((1, D)),
            full((D, D)), full((1, D)),
            full((1, D)), full((1, D)),
            full((D, D)), full((D, D)), full((1, D)),
        ],
        out_specs=pl.BlockSpec((RB, D), lambda i: (i, 0)),
        out_shape=jax.ShapeDtypeStruct((NUM_HE, D), f32),
    )(aggp, denp, hyperedge, att_row,
      ln0_g.reshape(1, D), ln0_b.reshape(1, D),
      W_rff, b_rff.reshape(1, D),
      ln1_g.reshape(1, D), ln1_b.reshape(1, D),
      W_upd[:D], W_upd[D:], b_upd.reshape(1, D))
    return out
